# baseline (speedup 1.0000x reference)
import functools

import numpy as np
import jax
import jax.numpy as jnp
from jax import lax
from jax.experimental import pallas as pl
from jax.experimental.pallas import tpu as pltpu
from jax.experimental.pallas import tpu_sc as plsc

_B, _L = 4, 2048
_D, _FF, _NH = 128, 512, 4
_MEM = 256 + 64
_K = _MEM + _L
_DH = _D // _NH
_BQ = 512
_NQ = _L // _BQ
_SQRTD = float(np.sqrt(float(_D)))
_ISQ = float(1.0 / np.sqrt(float(_DH)))

_INVFREQ = 1.0 / (10000.0 ** (np.arange(0, _D, 2, dtype=np.float64) / _D))
_ANG_I = (np.arange(_L, dtype=np.float64) + _MEM)[:, None] * _INVFREQ[None, :]
_SIN_I, _COS_I = np.sin(_ANG_I), np.cos(_ANG_I)
_TA = np.concatenate([_SIN_I, _SIN_I], axis=1).astype(np.float32)
_TB = np.concatenate([_COS_I, -_COS_I], axis=1).astype(np.float32)
_ANG_J = np.arange(_K, dtype=np.float64)[:, None] * _INVFREQ[None, :]
_WT = np.concatenate([np.cos(_ANG_J), np.sin(_ANG_J)], axis=1)
_WT_BF = _WT.astype(jnp.bfloat16)

_NROWS_IDX = (_B * _L) // 128
_NW = 32
_RPW = (_B * _L) // _NW


def _embed_gather(idx2d, table):
    mesh = plsc.VectorSubcoreMesh(core_axis_name="c", subcore_axis_name="s")

    @functools.partial(
        pl.kernel,
        mesh=mesh,
        out_type=jax.ShapeDtypeStruct((_B * _L // 2, _D), jnp.float32),
        scratch_types=[
            pltpu.VMEM((1, 128), jnp.int32),
            pltpu.VMEM((128, _D), jnp.float32),
            pltpu.SemaphoreType.DMA,
        ],
    )
    def gk(idx_hbm, table_hbm, out_hbm, idx_v, rows_v, sem):
        wid = lax.axis_index("s") * 2 + lax.axis_index("c")
        pltpu.sync_copy(idx_hbm.at[pl.ds(wid, 1)], idx_v)
        pltpu.async_copy(table_hbm.at[idx_v.at[0]], rows_v, sem).wait()
        pltpu.sync_copy(rows_v, out_hbm.at[pl.ds(wid * 128, 128)])

    return gk(idx2d, table)


def _ln_in(x, g, b):
    m = jnp.mean(x, axis=1, keepdims=True)
    xc = x - m
    v = jnp.mean(xc * xc, axis=1, keepdims=True)
    return g * xc / jnp.sqrt(v + 1e-5) + b


def _mega_body(x_ref, mem_ref, wqkv_ref, bqkv_ref, wr_ref, bc_ref, br_ref,
               ta_ref, tb_ref, wt_ref, wo_ref, bo_ref, g1_ref, be1_ref,
               w1_ref, bb1_ref, w2_ref, bb2_ref, g2_ref, be2_ref, out_ref):
    f32 = jnp.float32
    bf = jnp.bfloat16
    xs = x_ref[0] * _SQRTD
    mem = mem_ref[0]
    wqkv = wqkv_ref[...]
    q = jnp.dot(xs, wqkv[:, :_D], preferred_element_type=f32) + bqkv_ref[:, :_D]
    kx = jnp.dot(xs, wqkv[:, _D:2 * _D], preferred_element_type=f32) + bqkv_ref[:, _D:2 * _D]
    km = jnp.dot(mem, wqkv[:, _D:2 * _D], preferred_element_type=f32) + bqkv_ref[:, _D:2 * _D]
    vx = jnp.dot(xs, wqkv[:, 2 * _D:], preferred_element_type=f32) + bqkv_ref[:, 2 * _D:]
    vm = jnp.dot(mem, wqkv[:, 2 * _D:], preferred_element_type=f32) + bqkv_ref[:, 2 * _D:]
    phi = q + br_ref[...]
    ta = ta_ref[...]
    tb = tb_ref[...]
    wt = wt_ref[...]

    qe, ke, ve = [], [], []
    ones_col = jnp.ones((_K, 8), dtype=bf)
    for n in range(_NH):
        sl = slice(n * _DH, (n + 1) * _DH)
        g = lax.dot_general(phi[:, sl], wr_ref[:, sl],
                            (((1,), (1,)), ((), ())), preferred_element_type=f32)
        gsw = jnp.concatenate([g[:, _D // 2:], g[:, :_D // 2]], axis=1)
        u_n = (g * ta + gsw * tb) * _ISQ
        qu_n = (q[:, sl] + bc_ref[:, sl]) * _ISQ
        qe.append(jnp.concatenate([qu_n.astype(bf), u_n.astype(bf)], axis=1))
        kh = jnp.concatenate([km[:, sl], kx[:, sl]], axis=0).astype(bf)
        vh = jnp.concatenate([vm[:, sl], vx[:, sl]], axis=0).astype(bf)
        ke.append(jnp.concatenate([kh, wt], axis=1))
        ve.append(jnp.concatenate([vh, ones_col], axis=1))

    tri = (lax.broadcasted_iota(jnp.int32, (_BQ, _BQ), 1)
           <= lax.broadcasted_iota(jnp.int32, (_BQ, _BQ), 0))
    o_blocks = []
    for qi in range(_NQ):
        kq = qi * _BQ + _BQ + _MEM
        kl = kq - _BQ
        qs = slice(qi * _BQ, (qi + 1) * _BQ)
        o_heads = []
        for n in range(_NH):
            sL = lax.dot_general(qe[n][qs], ke[n][:kl], (((1,), (1,)), ((), ())),
                                 preferred_element_type=f32)
            sR = lax.dot_general(qe[n][qs], ke[n][kl:kq], (((1,), (1,)), ((), ())),
                                 preferred_element_type=f32)
            eL = jnp.exp(sL).astype(bf)
            eR = jnp.where(tri, jnp.exp(sR), 0.0).astype(bf)
            o1 = (lax.dot_general(eL, ve[n][:kl], (((1,), (0,)), ((), ())),
                                  preferred_element_type=f32)
                  + lax.dot_general(eR, ve[n][kl:kq], (((1,), (0,)), ((), ())),
                                    preferred_element_type=f32))
            o_heads.append(o1[:, :_DH] * (1.0 / o1[:, _DH:_DH + 1]))
        o_blocks.append(jnp.concatenate(o_heads, axis=1))
    ao = jnp.concatenate(o_blocks, axis=0)

    t = jnp.dot(ao.astype(bf), wo_ref[...], preferred_element_type=f32) + bo_ref[...]
    h1 = _ln_in(xs + t, g1_ref[...], be1_ref[...])
    hh = jnp.maximum(jnp.dot(h1, w1_ref[...], preferred_element_type=f32) + bb1_ref[...], 0.0)
    ff = jnp.dot(hh, w2_ref[...], preferred_element_type=f32) + bb2_ref[...]
    out_ref[0] = _ln_in(h1 + ff, g2_ref[...], be2_ref[...])


def _mega(x, memory, wqkv, bqkv, wr, bc, br, wo, bo, g1, be1, w1, bb1, w2,
          bb2, g2, be2, nb):
    full = lambda shp: pl.BlockSpec(shp, lambda b: (0,) * len(shp))
    return pl.pallas_call(
        _mega_body,
        grid=(nb,),
        in_specs=[
            pl.BlockSpec((1, _L, _D), lambda b: (b, 0, 0)),
            pl.BlockSpec((1, _MEM, _D), lambda b: (b, 0, 0)),
            full((_D, 3 * _D)),
            full((1, 3 * _D)),
            full((_D, _D)),
            full((1, _D)),
            full((1, _D)),
            full((_L, _D)),
            full((_L, _D)),
            full((_K, _D)),
            full((_D, _D)),
            full((1, _D)),
            full((1, _D)),
            full((1, _D)),
            full((_D, _FF)),
            full((1, _FF)),
            full((_FF, _D)),
            full((1, _D)),
            full((1, _D)),
            full((1, _D)),
        ],
        out_specs=pl.BlockSpec((1, _L, _D), lambda b: (b, 0, 0)),
        out_shape=jax.ShapeDtypeStruct((nb, _L, _D), jnp.float32),
    )(x, memory, wqkv, bqkv, wr, bc, br, _TA, _TB, _WT_BF, wo, bo, g1, be1,
      w1, bb1, w2, bb2, g2, be2)


def kernel(tokens, table, memory, kernel_qkv, bias_qkv, kernel_r, kernel_o,
           bias_o, bias_context, bias_relative, gamma1, beta1, w1, b1, w2, b2,
           gamma2, beta2):
    tok = tokens.astype(jnp.int32).reshape(2, _NROWS_IDX // 2, 128)
    r2 = lambda a: a.reshape(1, -1)
    outs = []
    for hb in range(2):
        xf = _embed_gather(tok[hb], table)
        x = xf.reshape(_B // 2, _L, _D)
        outs.append(_mega(x, memory[2 * hb:2 * hb + 2], kernel_qkv,
                          r2(bias_qkv), kernel_r, r2(bias_context),
                          r2(bias_relative), kernel_o, r2(bias_o), r2(gamma1),
                          r2(beta1), w1, r2(b1), w2, r2(b2), r2(gamma2),
                          r2(beta2), _B // 2))
    return jnp.concatenate(outs, axis=0)


# speedup vs baseline: 1.1984x; 1.1984x over previous
import functools

import numpy as np
import jax
import jax.numpy as jnp
from jax import lax
from jax.experimental import pallas as pl
from jax.experimental.pallas import tpu as pltpu
from jax.experimental.pallas import tpu_sc as plsc

_B, _L = 4, 2048
_D, _FF, _NH = 128, 512, 4
_MEM = 256 + 64
_K = _MEM + _L
_DH = _D // _NH
_BQ = 512
_NQ = _L // _BQ
_SQRTD = float(np.sqrt(float(_D)))
_ISQ = float(1.0 / np.sqrt(float(_DH)))

_INVFREQ = 1.0 / (10000.0 ** (np.arange(0, _D, 2, dtype=np.float64) / _D))
_ANG_I = (np.arange(_L, dtype=np.float64) + _MEM)[:, None] * _INVFREQ[None, :]
_SIN_I, _COS_I = np.sin(_ANG_I), np.cos(_ANG_I)
_TA = np.concatenate([_SIN_I, _SIN_I], axis=1).astype(np.float32)
_TB = np.concatenate([_COS_I, -_COS_I], axis=1).astype(np.float32)
_ANG_J = np.arange(_K, dtype=np.float64)[:, None] * _INVFREQ[None, :]
_WT = np.concatenate([np.cos(_ANG_J), np.sin(_ANG_J)], axis=1)
_WT_BF = _WT.astype(jnp.bfloat16)

_NROWS_IDX = (_B * _L) // 128
_NW = 32
_RPW = (_B * _L) // _NW


def _embed_gather(idx2d, table):
    mesh = plsc.VectorSubcoreMesh(core_axis_name="c", subcore_axis_name="s")

    @functools.partial(
        pl.kernel,
        mesh=mesh,
        out_type=jax.ShapeDtypeStruct((_B * _L, _D), jnp.float32),
        scratch_types=[
            pltpu.VMEM((2, 128), jnp.int32),
            pltpu.VMEM((_RPW, _D), jnp.float32),
            pltpu.SemaphoreType.DMA,
        ],
    )
    def gk(idx_hbm, table_hbm, out_hbm, idx_v, rows_v, sem):
        wid = lax.axis_index("s") * 2 + lax.axis_index("c")
        pltpu.sync_copy(idx_hbm.at[pl.ds(wid * 2, 2)], idx_v)
        c0 = pltpu.async_copy(table_hbm.at[idx_v.at[0]], rows_v.at[pl.ds(0, 128)], sem)
        c1 = pltpu.async_copy(table_hbm.at[idx_v.at[1]], rows_v.at[pl.ds(128, 128)], sem)
        c0.wait()
        c1.wait()
        pltpu.sync_copy(rows_v, out_hbm.at[pl.ds(wid * _RPW, _RPW)])

    return gk(idx2d, table)


def _ln_in(x, g, b):
    m = jnp.mean(x, axis=1, keepdims=True)
    xc = x - m
    v = jnp.mean(xc * xc, axis=1, keepdims=True)
    return g * xc / jnp.sqrt(v + 1e-5) + b


def _mega_body(x_ref, mem_ref, wqkv_ref, bqkv_ref, wr_ref, bc_ref, br_ref,
               ta_ref, tb_ref, wt_ref, wo_ref, bo_ref, g1_ref, be1_ref,
               w1_ref, bb1_ref, w2_ref, bb2_ref, g2_ref, be2_ref, out_ref):
    f32 = jnp.float32
    bf = jnp.bfloat16
    xs = x_ref[0] * _SQRTD
    mem = mem_ref[0]
    wqkv = wqkv_ref[...]
    q = jnp.dot(xs, wqkv[:, :_D], preferred_element_type=f32) + bqkv_ref[:, :_D]
    kx = jnp.dot(xs, wqkv[:, _D:2 * _D], preferred_element_type=f32) + bqkv_ref[:, _D:2 * _D]
    km = jnp.dot(mem, wqkv[:, _D:2 * _D], preferred_element_type=f32) + bqkv_ref[:, _D:2 * _D]
    vx = jnp.dot(xs, wqkv[:, 2 * _D:], preferred_element_type=f32) + bqkv_ref[:, 2 * _D:]
    vm = jnp.dot(mem, wqkv[:, 2 * _D:], preferred_element_type=f32) + bqkv_ref[:, 2 * _D:]
    phi = q + br_ref[...]
    ta = ta_ref[...]
    tb = tb_ref[...]
    wt = wt_ref[...]

    qe, ke, ve = [], [], []
    ones_col = jnp.ones((_K, 8), dtype=bf)
    for n in range(_NH):
        sl = slice(n * _DH, (n + 1) * _DH)
        g = lax.dot_general(phi[:, sl], wr_ref[:, sl],
                            (((1,), (1,)), ((), ())), preferred_element_type=f32)
        gsw = jnp.concatenate([g[:, _D // 2:], g[:, :_D // 2]], axis=1)
        u_n = (g * ta + gsw * tb) * _ISQ
        qu_n = (q[:, sl] + bc_ref[:, sl]) * _ISQ
        qe.append(jnp.concatenate([qu_n.astype(bf), u_n.astype(bf)], axis=1))
        kh = jnp.concatenate([km[:, sl], kx[:, sl]], axis=0).astype(bf)
        vh = jnp.concatenate([vm[:, sl], vx[:, sl]], axis=0).astype(bf)
        ke.append(jnp.concatenate([kh, wt], axis=1))
        ve.append(jnp.concatenate([vh, ones_col], axis=1))

    tri = (lax.broadcasted_iota(jnp.int32, (_BQ, _BQ), 1)
           <= lax.broadcasted_iota(jnp.int32, (_BQ, _BQ), 0))
    o_blocks = []
    for qi in range(_NQ):
        kq = qi * _BQ + _BQ + _MEM
        kl = kq - _BQ
        qs = slice(qi * _BQ, (qi + 1) * _BQ)
        o_heads = []
        for n in range(_NH):
            sL = lax.dot_general(qe[n][qs], ke[n][:kl], (((1,), (1,)), ((), ())),
                                 preferred_element_type=f32)
            sR = lax.dot_general(qe[n][qs], ke[n][kl:kq], (((1,), (1,)), ((), ())),
                                 preferred_element_type=f32)
            eL = jnp.exp(sL).astype(bf)
            eR = jnp.where(tri, jnp.exp(sR), 0.0).astype(bf)
            o1 = (lax.dot_general(eL, ve[n][:kl], (((1,), (0,)), ((), ())),
                                  preferred_element_type=f32)
                  + lax.dot_general(eR, ve[n][kl:kq], (((1,), (0,)), ((), ())),
                                    preferred_element_type=f32))
            o_heads.append(o1[:, :_DH] * (1.0 / o1[:, _DH:_DH + 1]))
        o_blocks.append(jnp.concatenate(o_heads, axis=1))
    ao = jnp.concatenate(o_blocks, axis=0)

    t = jnp.dot(ao.astype(bf), wo_ref[...], preferred_element_type=f32) + bo_ref[...]
    h1 = _ln_in(xs + t, g1_ref[...], be1_ref[...])
    hh = jnp.maximum(jnp.dot(h1, w1_ref[...], preferred_element_type=f32) + bb1_ref[...], 0.0)
    ff = jnp.dot(hh, w2_ref[...], preferred_element_type=f32) + bb2_ref[...]
    out_ref[0] = _ln_in(h1 + ff, g2_ref[...], be2_ref[...])


def _mega(x, memory, wqkv, bqkv, wr, bc, br, wo, bo, g1, be1, w1, bb1, w2,
          bb2, g2, be2, nb):
    full = lambda shp: pl.BlockSpec(shp, lambda b: (0,) * len(shp))
    return pl.pallas_call(
        _mega_body,
        grid=(nb,),
        in_specs=[
            pl.BlockSpec((1, _L, _D), lambda b: (b, 0, 0)),
            pl.BlockSpec((1, _MEM, _D), lambda b: (b, 0, 0)),
            full((_D, 3 * _D)),
            full((1, 3 * _D)),
            full((_D, _D)),
            full((1, _D)),
            full((1, _D)),
            full((_L, _D)),
            full((_L, _D)),
            full((_K, _D)),
            full((_D, _D)),
            full((1, _D)),
            full((1, _D)),
            full((1, _D)),
            full((_D, _FF)),
            full((1, _FF)),
            full((_FF, _D)),
            full((1, _D)),
            full((1, _D)),
            full((1, _D)),
        ],
        out_specs=pl.BlockSpec((1, _L, _D), lambda b: (b, 0, 0)),
        out_shape=jax.ShapeDtypeStruct((nb, _L, _D), jnp.float32),
    )(x, memory, wqkv, bqkv, wr, bc, br, _TA, _TB, _WT_BF, wo, bo, g1, be1,
      w1, bb1, w2, bb2, g2, be2)


def kernel(tokens, table, memory, kernel_qkv, bias_qkv, kernel_r, kernel_o,
           bias_o, bias_context, bias_relative, gamma1, beta1, w1, b1, w2, b2,
           gamma2, beta2):
    idx2d = tokens.astype(jnp.int32).reshape(_NROWS_IDX, 128)
    xf = _embed_gather(idx2d, table)
    x = xf.reshape(_B, _L, _D)
    r2 = lambda a: a.reshape(1, -1)
    return _mega(x, memory, kernel_qkv, r2(bias_qkv), kernel_r,
                 r2(bias_context), r2(bias_relative), kernel_o, r2(bias_o),
                 r2(gamma1), r2(beta1), w1, r2(b1), w2, r2(b2), r2(gamma2),
                 r2(beta2), _B)
